# baseline (speedup 1.0000x reference)
import functools

import jax
import jax.numpy as jnp
from jax import lax
from jax.experimental import pallas as pl
from jax.experimental.pallas import tpu as pltpu
from jax.experimental.pallas import tpu_sc as plsc

N = 10000
NP = 10112
E = 320000
EP = 327680
H = 64
T1W = 64
T2W = 64

NC, NS = 2, 16
NW = NC * NS
ROWS_PER_TILE_SC = NP // NS
EC = 64
IDX_ROWS = EP // EC
GRP = 8

_LEAK = 0.2


def _make_sc_round(dw, feature_split):
  mesh = plsc.VectorSubcoreMesh(
      core_axis_name="c", subcore_axis_name="s", num_cores=NC, num_subcores=NS)
  phase_rows = 160
  phases = 2 if feature_split else 1
  bank = 4
  n_groups = phase_rows // bank

  cnt_types = []
  if feature_split:
    cnt_types = [
        pltpu.VMEM((EC, 16), jnp.float32),
        pltpu.VMEM_SHARED((NP, 16), jnp.float32),
        pltpu.SemaphoreType.DMA,
    ]
  out_t = jax.ShapeDtypeStruct((NC, NP, dw), jnp.float32)
  if feature_split:
    out_t = (out_t, jax.ShapeDtypeStruct((NC, NP, 16), jnp.float32))

  @functools.partial(
      pl.kernel,
      out_type=out_t,
      mesh=mesh,
      compiler_params=pltpu.CompilerParams(use_tc_tiling_on_sc=False),
      scratch_types=[
          pltpu.VMEM((phase_rows, EC), jnp.int32),
          pltpu.VMEM((phase_rows, EC), jnp.int32),
          pltpu.VMEM((2 * bank, EC, dw), jnp.float32),
          pltpu.VMEM_SHARED((NP, dw), jnp.float32),
          pltpu.SemaphoreType.DMA,
          pltpu.SemaphoreType.DMA,
      ] + cnt_types,
  )
  def sc_round(table_hbm, src_hbm, dst_hbm, zeros_hbm, *rest):
    if feature_split:
      (onesz_hbm, out_hbm, cnt_hbm,
       sidx_v, didx_v, bufs_v, acc_sh, sem_g, sem_s,
       ones_v, cnt_sh, sem_c) = rest
    else:
      (out_hbm, sidx_v, didx_v, bufs_v, acc_sh, sem_g, sem_s) = rest
    c = lax.axis_index("c")
    s = lax.axis_index("s")

    pltpu.sync_copy(
        zeros_hbm, acc_sh.at[pl.ds(s * ROWS_PER_TILE_SC, ROWS_PER_TILE_SC)])
    if feature_split:
      pltpu.sync_copy(onesz_hbm.at[pl.ds(0, EC)], ones_v)
      pltpu.sync_copy(
          onesz_hbm.at[pl.ds(EC, ROWS_PER_TILE_SC)],
          cnt_sh.at[pl.ds(s * ROWS_PER_TILE_SC, ROWS_PER_TILE_SC)])
    plsc.subcore_barrier()

    tbl = table_hbm.at[c] if feature_split else table_hbm

    def start_g(g, b):
      for k in range(bank):
        pltpu.async_copy(
            tbl.at[sidx_v.at[bank * g + k]], bufs_v.at[bank * b + k], sem_g)

    def drain_g(b):
      for k in range(bank):
        pltpu.make_async_copy(
            tbl.at[pl.ds(0, EC)], bufs_v.at[bank * b + k], sem_g).wait()

    def start_s(g, b):
      for k in range(bank):
        pltpu.async_copy(
            bufs_v.at[bank * b + k], acc_sh.at[didx_v.at[bank * g + k]],
            sem_s, add=True)
      if feature_split:
        @pl.when(c == cnt_phase[0])
        def _():
          for k in range(bank):
            pltpu.async_copy(
                ones_v, cnt_sh.at[didx_v.at[bank * g + k]], sem_c, add=True)

    def drain_s(b):
      for k in range(bank):
        pltpu.make_async_copy(
            zeros_hbm.at[pl.ds(0, EC)], bufs_v.at[bank * b + k], sem_s).wait()
      if feature_split:
        @pl.when(c == cnt_phase[0])
        def _():
          for k in range(bank):
            pltpu.make_async_copy(
                onesz_hbm.at[pl.ds(0, EC)], ones_v, sem_c).wait()

    cnt_phase = [0]
    for h in range(phases):
      cnt_phase[0] = h
      if feature_split:
        base = s * (phases * phase_rows) + h * phase_rows
      else:
        base = (s * NC + c) * phase_rows
      pltpu.sync_copy(src_hbm.at[pl.ds(base, phase_rows)], sidx_v)
      pltpu.sync_copy(dst_hbm.at[pl.ds(base, phase_rows)], didx_v)

      start_g(0, 0)
      drain_g(0)
      start_s(0, 0)
      start_g(1, 1)

      def pair_body(p, carry):
        for b in (1, 0):
          g = 2 * p + (1 if b == 1 else 2)
          drain_g(b)
          start_s(g, b)
          drain_s(1 - b)
          start_g(g + 1, 1 - b)
        return carry

      lax.fori_loop(0, (n_groups - 2) // 2, pair_body, 0)

      drain_g(1)
      start_s(n_groups - 1, 1)
      drain_s(0)
      drain_s(1)

    plsc.subcore_barrier()
    pltpu.sync_copy(
        acc_sh.at[pl.ds(s * ROWS_PER_TILE_SC, ROWS_PER_TILE_SC)],
        out_hbm.at[c, pl.ds(s * ROWS_PER_TILE_SC, ROWS_PER_TILE_SC), :])
    if feature_split:
      pltpu.sync_copy(
          cnt_sh.at[pl.ds(s * ROWS_PER_TILE_SC, ROWS_PER_TILE_SC)],
          cnt_hbm.at[c, pl.ds(s * ROWS_PER_TILE_SC, ROWS_PER_TILE_SC), :])

  return sc_round


_sc_cache = {}


def _sc_round(which):
  if which not in _sc_cache:
    _sc_cache[which] = (_make_sc_round(T1W, True) if which == 1
                        else _make_sc_round(T2W, False))
  return _sc_cache[which]


def _leaky(x):
  return jnp.where(x >= 0, x, _LEAK * x)


def _bf16_dot(a, b):
  return jnp.dot(a.astype(jnp.bfloat16), b.astype(jnp.bfloat16),
                 preferred_element_type=jnp.float32)


def _row_mask():
  rows = lax.broadcasted_iota(jnp.int32, (NP, 1), 0)
  return (rows < N).astype(jnp.float32)


def _bn_leaky(h, g, b):
  mask = _row_mask()
  hm = h * mask
  m = jnp.sum(hm, axis=0, keepdims=True) * (1.0 / N)
  v = jnp.sum(hm * hm, axis=0, keepdims=True) * (1.0 / N) - m * m
  hb = g[None, :] * (h - m) * lax.rsqrt(v + 1e-5) + b[None, :]
  return _leaky(hb)


def _tc_combine1(x_pad, agg1, cnt16, b1, g1, be1, w1t):
  def body(x_ref, agg_ref, cnt_ref, b1_ref, g1_ref, be1_ref, w1t_ref, o_ref):
    agg = jnp.concatenate([agg_ref[0], agg_ref[1]], axis=1)
    cnt = cnt_ref[0, :, 0:1] + cnt_ref[1, :, 0:1]
    rden = 1.0 / (1.0 + jnp.maximum(cnt, 1.0))
    t = (x_ref[...] + agg) * rden
    h = _leaky(_bf16_dot(t, w1t_ref[...]) + b1_ref[...][None, :])
    hb = _bn_leaky(h, g1_ref[...], be1_ref[...])
    o_ref[...] = hb * _row_mask()

  return pl.pallas_call(
      body, out_shape=jax.ShapeDtypeStruct((NP, T2W), jnp.float32))(
          x_pad, agg1, cnt16, b1, g1, be1, w1t)


def _tc_combine2(table2, acc2, cnt16, b2, g2, be2, w2t, wc1t, bc1, wc2t, bc2):
  def body(t2_ref, acc2_ref, cnt_ref, b2_ref, g2_ref, be2_ref, w2t_ref,
           wc1t_ref, bc1_ref, wc2t_ref, bc2_ref, o_ref):
    agg = acc2_ref[0] + acc2_ref[1]
    cnt = cnt_ref[0, :, 0:1] + cnt_ref[1, :, 0:1]
    rden = 1.0 / (1.0 + jnp.maximum(cnt, 1.0))
    t = (t2_ref[...] + agg) * rden
    h = _leaky(_bf16_dot(t, w2t_ref[...]) + b2_ref[...][None, :])
    hb = _bn_leaky(h, g2_ref[...], be2_ref[...])
    hc = jnp.maximum(_bf16_dot(hb, wc1t_ref[...]) + bc1_ref[...][None, :], 0.0)
    logits = _bf16_dot(hc, wc2t_ref[...])
    o_ref[...] = logits + bc2_ref[...][None, :]

  return pl.pallas_call(
      body, out_shape=jax.ShapeDtypeStruct((NP, 128), jnp.float32))(
          table2, acc2, cnt16, b2, g2, be2, w2t, wc1t, bc1, wc2t, bc2)


def kernel(x, edge_index, W1, b1, g1, be1, W2, b2, g2, be2, Wc1, bc1, Wc2, bc2):
  x_pad = jnp.zeros((NP, 128), jnp.float32).at[:N].set(x)
  table1 = jnp.stack([x_pad[:, 0:H], x_pad[:, H:128]])
  onesz = jnp.zeros((EC + ROWS_PER_TILE_SC, 16), jnp.float32).at[:EC].set(1.0)
  src = edge_index[0]
  dst = edge_index[1]
  pad = jnp.full((EP - E,), N, jnp.int32)
  src2d = jnp.concatenate([src, pad]).reshape(IDX_ROWS, EC)
  dst2d = jnp.concatenate([dst, pad]).reshape(IDX_ROWS, EC)
  zeros1 = jnp.zeros((ROWS_PER_TILE_SC, T1W), jnp.float32)
  zeros2 = jnp.zeros((ROWS_PER_TILE_SC, T2W), jnp.float32)
  w1t = W1.T
  w2t = W2.T
  wc1t = jnp.zeros((H, 128), jnp.float32).at[:, :32].set(Wc1.T)
  bc1p = jnp.zeros((128,), jnp.float32).at[:32].set(bc1)
  wc2t = jnp.zeros((128, 128), jnp.float32).at[:32, :2].set(Wc2.T)
  bc2p = jnp.zeros((128,), jnp.float32).at[:2].set(bc2)

  agg1, cnt16 = _sc_round(1)(table1, src2d, dst2d, zeros1, onesz)
  table2 = _tc_combine1(x_pad, agg1, cnt16, b1, g1, be1, w1t)

  acc2 = _sc_round(2)(table2, src2d, dst2d, zeros2)
  out = _tc_combine2(table2, acc2, cnt16, b2, g2, be2, w2t, wc1t, bc1p,
                     wc2t, bc2p)

  return out[:N, :2]


# speedup vs baseline: 1.0389x; 1.0389x over previous
import functools

import jax
import jax.numpy as jnp
from jax import lax
from jax.experimental import pallas as pl
from jax.experimental.pallas import tpu as pltpu
from jax.experimental.pallas import tpu_sc as plsc

N = 10000
NP = 10112
E = 320000
EP = 327680
H = 64
T1W = 64
T2W = 64

NC, NS = 2, 16
NW = NC * NS
ROWS_PER_TILE_SC = NP // NS
EC = 64
IDX_ROWS = EP // EC
GRP = 8

_LEAK = 0.2


def _make_sc_round(dw, feature_split):
  mesh = plsc.VectorSubcoreMesh(
      core_axis_name="c", subcore_axis_name="s", num_cores=NC, num_subcores=NS)
  phase_rows = 160
  phases = 2 if feature_split else 1
  bank = 4
  n_groups = phase_rows // bank

  cnt_types = []
  if feature_split:
    cnt_types = [
        pltpu.VMEM((EC, 16), jnp.float32),
        pltpu.VMEM_SHARED((NP, 16), jnp.float32),
        pltpu.SemaphoreType.DMA,
    ]
  out_t = jax.ShapeDtypeStruct((NC, NP, dw), jnp.float32)
  if feature_split:
    out_t = (out_t, jax.ShapeDtypeStruct((NP, 16), jnp.float32))

  @functools.partial(
      pl.kernel,
      out_type=out_t,
      mesh=mesh,
      compiler_params=pltpu.CompilerParams(use_tc_tiling_on_sc=False),
      scratch_types=[
          pltpu.VMEM((phase_rows, EC), jnp.int32),
          pltpu.VMEM((phase_rows, EC), jnp.int32),
          pltpu.VMEM((2 * bank, EC, dw), jnp.float32),
          pltpu.VMEM_SHARED((NP, dw), jnp.float32),
          pltpu.SemaphoreType.DMA,
          pltpu.SemaphoreType.DMA,
      ] + cnt_types,
  )
  def sc_round(table_hbm, src_hbm, dst_hbm, zeros_hbm, *rest):
    if feature_split:
      (onesz_hbm, out_hbm, cnt_hbm,
       sidx_v, didx_v, bufs_v, acc_sh, sem_g, sem_s,
       ones_v, cnt_sh, sem_c) = rest
    else:
      (out_hbm, sidx_v, didx_v, bufs_v, acc_sh, sem_g, sem_s) = rest
    c = lax.axis_index("c")
    s = lax.axis_index("s")

    pltpu.sync_copy(
        zeros_hbm, acc_sh.at[pl.ds(s * ROWS_PER_TILE_SC, ROWS_PER_TILE_SC)])
    if feature_split:
      pltpu.sync_copy(onesz_hbm.at[pl.ds(0, EC)], ones_v)
      with_c0 = pl.when(c == 0)
      @with_c0
      def _():
        pltpu.sync_copy(
            onesz_hbm.at[pl.ds(EC, ROWS_PER_TILE_SC)],
            cnt_sh.at[pl.ds(s * ROWS_PER_TILE_SC, ROWS_PER_TILE_SC)])
    plsc.subcore_barrier()

    tbl = table_hbm.at[c] if feature_split else table_hbm

    def start_g(g, b):
      for k in range(bank):
        pltpu.async_copy(
            tbl.at[sidx_v.at[bank * g + k]], bufs_v.at[bank * b + k], sem_g)

    def drain_g(b):
      for k in range(bank):
        pltpu.make_async_copy(
            tbl.at[pl.ds(0, EC)], bufs_v.at[bank * b + k], sem_g).wait()

    def start_s(g, b):
      for k in range(bank):
        pltpu.async_copy(
            bufs_v.at[bank * b + k], acc_sh.at[didx_v.at[bank * g + k]],
            sem_s, add=True)
      if feature_split:
        @pl.when(c == 0)
        def _():
          for k in range(bank):
            pltpu.async_copy(
                ones_v, cnt_sh.at[didx_v.at[bank * g + k]], sem_c, add=True)

    def drain_s(b):
      for k in range(bank):
        pltpu.make_async_copy(
            zeros_hbm.at[pl.ds(0, EC)], bufs_v.at[bank * b + k], sem_s).wait()
      if feature_split:
        @pl.when(c == 0)
        def _():
          for k in range(bank):
            pltpu.make_async_copy(
                onesz_hbm.at[pl.ds(0, EC)], ones_v, sem_c).wait()

    for h in range(phases):
      if feature_split:
        base = s * (phases * phase_rows) + h * phase_rows
      else:
        base = (s * NC + c) * phase_rows
      pltpu.sync_copy(src_hbm.at[pl.ds(base, phase_rows)], sidx_v)
      pltpu.sync_copy(dst_hbm.at[pl.ds(base, phase_rows)], didx_v)

      start_g(0, 0)
      drain_g(0)
      start_s(0, 0)
      start_g(1, 1)

      def pair_body(p, carry):
        for b in (1, 0):
          g = 2 * p + (1 if b == 1 else 2)
          drain_g(b)
          start_s(g, b)
          drain_s(1 - b)
          start_g(g + 1, 1 - b)
        return carry

      lax.fori_loop(0, (n_groups - 2) // 2, pair_body, 0)

      drain_g(1)
      start_s(n_groups - 1, 1)
      drain_s(0)
      drain_s(1)

    plsc.subcore_barrier()
    pltpu.sync_copy(
        acc_sh.at[pl.ds(s * ROWS_PER_TILE_SC, ROWS_PER_TILE_SC)],
        out_hbm.at[c, pl.ds(s * ROWS_PER_TILE_SC, ROWS_PER_TILE_SC), :])
    if feature_split:
      @pl.when(c == 0)
      def _():
        pltpu.sync_copy(
            cnt_sh.at[pl.ds(s * ROWS_PER_TILE_SC, ROWS_PER_TILE_SC)],
            cnt_hbm.at[pl.ds(s * ROWS_PER_TILE_SC, ROWS_PER_TILE_SC), :])

  return sc_round


_sc_cache = {}


def _sc_round(which):
  if which not in _sc_cache:
    _sc_cache[which] = (_make_sc_round(T1W, True) if which == 1
                        else _make_sc_round(T2W, False))
  return _sc_cache[which]


def _leaky(x):
  return jnp.where(x >= 0, x, _LEAK * x)


def _bf16_dot(a, b):
  return jnp.dot(a.astype(jnp.bfloat16), b.astype(jnp.bfloat16),
                 preferred_element_type=jnp.float32)


def _row_mask():
  rows = lax.broadcasted_iota(jnp.int32, (NP, 1), 0)
  return (rows < N).astype(jnp.float32)


def _bn_leaky(h, g, b):
  mask = _row_mask()
  hm = h * mask
  m = jnp.sum(hm, axis=0, keepdims=True) * (1.0 / N)
  v = jnp.sum(hm * hm, axis=0, keepdims=True) * (1.0 / N) - m * m
  hb = g[None, :] * (h - m) * lax.rsqrt(v + 1e-5) + b[None, :]
  return _leaky(hb)


def _tc_combine1(x_pad, agg1, cnt16, b1, g1, be1, w1t):
  def body(x_ref, agg_ref, cnt_ref, b1_ref, g1_ref, be1_ref, w1t_ref, o_ref):
    agg = jnp.concatenate([agg_ref[0], agg_ref[1]], axis=1)
    cnt = cnt_ref[:, 0:1]
    rden = 1.0 / (1.0 + jnp.maximum(cnt, 1.0))
    t = (x_ref[...] + agg) * rden
    h = _leaky(_bf16_dot(t, w1t_ref[...]) + b1_ref[...][None, :])
    hb = _bn_leaky(h, g1_ref[...], be1_ref[...])
    o_ref[...] = hb * _row_mask()

  return pl.pallas_call(
      body, out_shape=jax.ShapeDtypeStruct((NP, T2W), jnp.float32))(
          x_pad, agg1, cnt16, b1, g1, be1, w1t)


def _tc_combine2(table2, acc2, cnt16, b2, g2, be2, w2t, wc1t, bc1, wc2t, bc2):
  def body(t2_ref, acc2_ref, cnt_ref, b2_ref, g2_ref, be2_ref, w2t_ref,
           wc1t_ref, bc1_ref, wc2t_ref, bc2_ref, o_ref):
    agg = acc2_ref[0] + acc2_ref[1]
    cnt = cnt_ref[:, 0:1]
    rden = 1.0 / (1.0 + jnp.maximum(cnt, 1.0))
    t = (t2_ref[...] + agg) * rden
    h = _leaky(_bf16_dot(t, w2t_ref[...]) + b2_ref[...][None, :])
    hb = _bn_leaky(h, g2_ref[...], be2_ref[...])
    hc = jnp.maximum(_bf16_dot(hb, wc1t_ref[...]) + bc1_ref[...][None, :], 0.0)
    logits = _bf16_dot(hc, wc2t_ref[...])
    o_ref[...] = logits + bc2_ref[...][None, :]

  return pl.pallas_call(
      body, out_shape=jax.ShapeDtypeStruct((NP, 128), jnp.float32))(
          table2, acc2, cnt16, b2, g2, be2, w2t, wc1t, bc1, wc2t, bc2)


def kernel(x, edge_index, W1, b1, g1, be1, W2, b2, g2, be2, Wc1, bc1, Wc2, bc2):
  x_pad = jnp.zeros((NP, 128), jnp.float32).at[:N].set(x)
  table1 = jnp.stack([x_pad[:, 0:H], x_pad[:, H:128]])
  onesz = jnp.zeros((EC + ROWS_PER_TILE_SC, 16), jnp.float32).at[:EC].set(1.0)
  src = edge_index[0]
  dst = edge_index[1]
  pad = jnp.full((EP - E,), N, jnp.int32)
  src2d = jnp.concatenate([src, pad]).reshape(IDX_ROWS, EC)
  dst2d = jnp.concatenate([dst, pad]).reshape(IDX_ROWS, EC)
  zeros1 = jnp.zeros((ROWS_PER_TILE_SC, T1W), jnp.float32)
  zeros2 = jnp.zeros((ROWS_PER_TILE_SC, T2W), jnp.float32)
  w1t = W1.T
  w2t = W2.T
  wc1t = jnp.zeros((H, 128), jnp.float32).at[:, :32].set(Wc1.T)
  bc1p = jnp.zeros((128,), jnp.float32).at[:32].set(bc1)
  wc2t = jnp.zeros((128, 128), jnp.float32).at[:32, :2].set(Wc2.T)
  bc2p = jnp.zeros((128,), jnp.float32).at[:2].set(bc2)

  agg1, cnt16 = _sc_round(1)(table1, src2d, dst2d, zeros1, onesz)
  table2 = _tc_combine1(x_pad, agg1, cnt16, b1, g1, be1, w1t)

  acc2 = _sc_round(2)(table2, src2d, dst2d, zeros2)
  out = _tc_combine2(table2, acc2, cnt16, b2, g2, be2, w2t, wc1t, bc1p,
                     wc2t, bc2p)

  return out[:N, :2]
